# baseline (speedup 1.0000x reference)
import functools

import jax
import jax.numpy as jnp
from jax import lax
from jax.experimental import pallas as pl
from jax.experimental.pallas import tpu as pltpu
from jax.experimental.pallas import tpu_sc as plsc

F32 = jnp.float32
I32 = jnp.int32

N = 10000
S = 16
M = N * S
E = 320000
D = 128
H = 16
DH = D + H

EPAD = 327680
STRIPE = EPAD // 16
EBLK = 2048
MB = 1280
GRID_M = M // MB
SUBB = MB // S
DW = DH + 32
NCHUNK = 25
R = M // NCHUNK
HM = M // 2

_SC_PARAMS = pltpu.CompilerParams(
    needs_layout_passes=False, use_tc_tiling_on_sc=False)
_SC_PARAMS_TILED = pltpu.CompilerParams(needs_layout_passes=False)


def _mesh():
    return plsc.VectorSubcoreMesh(core_axis_name="c", subcore_axis_name="s")


def _tables_body(x_ref, ht_ref, w_ref, gcw_ref, gcb_ref, gsw_ref, gsb_ref,
                 gxw_ref, gxb_ref, xw_ref, hopw_ref, gct_ref, gst_ref,
                 gxt_ref):
    w = w_ref[...]
    xw_ref[...] = jnp.dot(x_ref[...], w[0:D, :], preferred_element_type=F32)
    ht = ht_ref[...]
    hw = jnp.dot(ht, w[D:DH, :], preferred_element_type=F32)
    hopw_ref[...] = jnp.concatenate([hw, jnp.zeros((12, DH), F32)], axis=0)
    gct_ref[...] = jax.nn.sigmoid(
        jnp.dot(ht, gcw_ref[...], preferred_element_type=F32) + gcb_ref[...])
    gst_ref[...] = jax.nn.sigmoid(
        jnp.dot(ht, gsw_ref[...], preferred_element_type=F32) + gsb_ref[...])
    gxt_ref[...] = jax.nn.sigmoid(
        jnp.dot(ht, gxw_ref[...], preferred_element_type=F32) + gxb_ref[...])


def _dinv_body(deg_ref, dinv_ref):
    dinv_ref[...] = lax.rsqrt(deg_ref[...][:, 0:1] + 1.0)


def _stats_body(agg_ref, dinv_ref, hopw_ref, out_ref):
    @pl.when(pl.program_id(0) == 0)
    def _():
        out_ref[...] = jnp.zeros_like(out_ref)
    agg = agg_ref[...]
    hb = dinv_ref[...] * (
        agg[:, 0:DH]
        + jnp.dot(agg[:, DH:DW], hopw_ref[...], preferred_element_type=F32))
    out_ref[0:1, :] += jnp.sum(hb, axis=0, keepdims=True)
    out_ref[1:2, :] += jnp.sum(hb * hb, axis=0, keepdims=True)


def _main_body(agg_ref, dinv_ref, hop_ref, stats_ref, hopw_ref,
               gamma_ref, beta_ref, w1_ref, b1_ref, w2_ref, b2_ref, subw_ref,
               subb_ref, ctxw_ref, ctxb_ref, gct_ref, gst_ref, gxt_ref,
               ctxall_ref, outp_ref):
    agg = agg_ref[...]
    hb = dinv_ref[...] * (
        agg[:, 0:DH]
        + jnp.dot(agg[:, DH:DW], hopw_ref[...], preferred_element_type=F32))
    stats = stats_ref[...]
    mu = stats[0:1, :] * (1.0 / M)
    var = stats[1:2, :] * (1.0 / M) - mu * mu
    xn = (hb - mu) * (gamma_ref[...] * lax.rsqrt(var + 1e-5)) + beta_ref[...]
    xn = jnp.maximum(xn, 0.0)
    t = jnp.maximum(jnp.dot(xn, w1_ref[...], preferred_element_type=F32)
                    + b1_ref[...], 0.0)
    h = jnp.dot(t, w2_ref[...], preferred_element_type=F32) + b2_ref[...]
    hop = hop_ref[...] + 1
    oh = (hop == lax.broadcasted_iota(I32, (MB, 20), 1)).astype(F32)
    gs = jnp.dot(oh, gst_ref[...], preferred_element_type=F32)
    gx = jnp.dot(oh, gxt_ref[...], preferred_element_type=F32)
    suball = (jnp.dot(h, subw_ref[...], preferred_element_type=F32)
              + subb_ref[...]) * gs
    ctxall_ref[...] = (jnp.dot(h, ctxw_ref[...], preferred_element_type=F32)
                       + ctxb_ref[...]) * gx
    r = lax.broadcasted_iota(I32, (SUBB, MB), 1)
    i = lax.broadcasted_iota(I32, (SUBB, MB), 0)
    pool = ((r // S) == i).astype(F32)
    subm = jnp.dot(pool, suball, preferred_element_type=F32) * (1.0 / S)
    csel = (r == i * S).astype(F32)
    hc = jnp.dot(csel, h, preferred_element_type=F32)
    ohc = jnp.dot(csel, oh, preferred_element_type=F32)
    gc = jnp.dot(ohc, gct_ref[...], preferred_element_type=F32)
    outp_ref[...] = subm + hc * gc


def _final_body(outp_ref, ctxp_ref, cnt0_ref, cnt1_ref, out_ref):
    ctxp = ctxp_ref[...]
    cnt = cnt0_ref[...][:, 0:1] + cnt1_ref[...][:, 0:1]
    out_ref[...] = outp_ref[...] + (ctxp[0] + ctxp[1]) / jnp.maximum(cnt, 1.0)


def _deg_cnt_kernel(dst_flat, map_flat, ones16, zeros16):
    @functools.partial(
        pl.kernel, mesh=_mesh(), compiler_params=_SC_PARAMS,
        out_type=(jax.ShapeDtypeStruct((HM, 16), F32),
                  jax.ShapeDtypeStruct((HM, 16), F32),
                  jax.ShapeDtypeStruct((N, 16), F32),
                  jax.ShapeDtypeStruct((N, 16), F32)),
        scratch_types=[
            pltpu.VMEM((EBLK,), I32),
            pltpu.VMEM((2304,), I32),
            pltpu.VMEM((1, 128), I32),
            pltpu.VMEM((128, 16), F32),
            pltpu.VMEM_SHARED((HM + 16, 16), F32),
        ],
    )
    def k(dst_hbm, map_hbm, ones_hbm, zero_hbm, d0_hbm, d1_hbm, c0_hbm,
          c1_hbm, st_d, bdst, idst, ones_v, accum):
        c = lax.axis_index("c")
        s = lax.axis_index("s")
        lo = c * HM
        trash = HM + lax.rem(s, 8)
        pltpu.sync_copy(ones_hbm, ones_v)
        pltpu.sync_copy(zero_hbm, accum.at[pl.ds(s * 5000, 5000)])

        @pl.when(s == 0)
        def _():
            pltpu.sync_copy(zero_hbm.at[pl.ds(0, 16)],
                            accum.at[pl.ds(HM, 16)])

        plsc.subcore_barrier()

        def block(b, cnt):
            pltpu.sync_copy(
                dst_hbm.at[pl.ds(s * STRIPE + b * EBLK, EBLK)], st_d)

            def fbody(t, cnt):
                dv = st_d[pl.ds(t * 16, 16)]
                m = (dv >= lo) & (dv < lo + HM)
                plsc.store_compressed(bdst.at[pl.ds(cnt, 16)], dv - lo,
                                      mask=m)
                return cnt + plsc.all_reduce_population_count(m)[0]
            cnt = lax.fori_loop(0, EBLK // 16, fbody, cnt)

            def flush(f, _):
                def pack(kk, _):
                    idst[0, pl.ds(kk * 16, 16)] = bdst[
                        pl.ds(f * 128 + kk * 16, 16)]
                    return 0
                lax.fori_loop(0, 8, pack, 0)
                pltpu.sync_copy(ones_v, accum.at[idst.at[0]], add=True)
                return 0
            nf = lax.shift_right_logical(cnt, 7)
            lax.fori_loop(0, nf, flush, 0)
            rem = lax.bitwise_and(cnt, 127)

            def mv(kk, _):
                bdst[pl.ds(kk * 16, 16)] = bdst[pl.ds(nf * 128 + kk * 16, 16)]
                return 0
            lax.fori_loop(0, 8, mv, 0)
            return rem
        cnt = lax.fori_loop(0, STRIPE // EBLK, block, jnp.int32(0))

        def pad(kk, _):
            bdst[pl.ds(cnt + kk * 16, 16)] = jnp.zeros((16,), I32) + trash
            return 0
        lax.fori_loop(0, 8, pad, 0)

        def pack2(kk, _):
            idst[0, pl.ds(kk * 16, 16)] = bdst[pl.ds(kk * 16, 16)]
            return 0

        @pl.when(cnt > 0)
        def _():
            lax.fori_loop(0, 8, pack2, 0)
            pltpu.sync_copy(ones_v, accum.at[idst.at[0]], add=True)

        plsc.subcore_barrier()

        @pl.when(c == 0)
        def _():
            pltpu.sync_copy(accum.at[pl.ds(s * 5000, 5000)],
                            d0_hbm.at[pl.ds(s * 5000, 5000)])

        @pl.when(c == 1)
        def _():
            pltpu.sync_copy(accum.at[pl.ds(s * 5000, 5000)],
                            d1_hbm.at[pl.ds(s * 5000, 5000)])

        plsc.subcore_barrier()

        def zc(kk, _):
            pltpu.sync_copy(zero_hbm.at[pl.ds(0, 625)],
                            accum.at[pl.ds(s * 625, 625)])
            return 0
        lax.fori_loop(0, 1, zc, 0)
        plsc.subcore_barrier()
        nbm = 39 + jnp.where(s == 0, 1, 0)
        bbase = s * 39 + jnp.where(s > 0, 1, 0)

        def mblock(i, _):
            boff = c * HM + (bbase + i) * 128
            pltpu.sync_copy(map_hbm.at[pl.ds(boff, 128)], idst.at[0])
            pltpu.sync_copy(ones_v, accum.at[idst.at[0]], add=True)
            return 0
        lax.fori_loop(0, nbm, mblock, 0)
        plsc.subcore_barrier()

        @pl.when(c == 0)
        def _():
            pltpu.sync_copy(accum.at[pl.ds(s * 625, 625)],
                            c0_hbm.at[pl.ds(s * 625, 625)])

        @pl.when(c == 1)
        def _():
            pltpu.sync_copy(accum.at[pl.ds(s * 625, 625)],
                            c1_hbm.at[pl.ds(s * 625, 625)])

    return k(dst_flat, map_flat, ones16, zeros16)


def _ya_kernel(xw_x, map_flat, dinv_flat, hop_flat):
    @functools.partial(
        pl.kernel, mesh=_mesh(), compiler_params=_SC_PARAMS,
        out_type=jax.ShapeDtypeStruct((M, DW), F32),
        scratch_types=[
            pltpu.VMEM((1, 128), I32),
            pltpu.VMEM((128,), F32),
            pltpu.VMEM((128,), I32),
            pltpu.VMEM((128, DH), F32),
            pltpu.VMEM((128, DW), F32),
            pltpu.SemaphoreType.DMA,
        ],
    )
    def k(xw_hbm, map_hbm, dinv_hbm, hop_hbm, y_hbm, irow, dstage, hstage,
          rows_v, ybuf, sem):
        c = lax.axis_index("c")
        s = lax.axis_index("s")
        w = s * 2 + c
        nb = 39 + jnp.where(w < 2, 1, 0)
        lane = lax.iota(I32, 16)

        def body(i, _):
            boff = (w + 32 * i) * 128
            pltpu.sync_copy(map_hbm.at[pl.ds(boff, 128)], irow.at[0])
            pltpu.async_copy(xw_hbm.at[irow.at[0]], rows_v, sem).wait()
            pltpu.sync_copy(dinv_hbm.at[pl.ds(boff, 128)], dstage)
            pltpu.sync_copy(hop_hbm.at[pl.ds(boff, 128)], hstage)

            def rbody(g, _):
                dvec = dstage[pl.ds(g * 16, 16)]
                hvec = hstage[pl.ds(g * 16, 16)] + 1
                for r0 in range(16):
                    r = g * 16 + r0
                    d = dvec[r0]
                    h = hvec[r0]
                    for kk in range(DH // 16):
                        ybuf[r, pl.ds(kk * 16, 16)] = (
                            rows_v[r, pl.ds(kk * 16, 16)] * d)
                    ybuf[r, pl.ds(DH, 16)] = jnp.where(lane == h, d, 0.0)
                    ybuf[r, pl.ds(DH + 16, 16)] = jnp.where(
                        lane == h - 16, d, 0.0)
                return 0
            lax.fori_loop(0, 8, rbody, 0)
            pltpu.sync_copy(ybuf, y_hbm.at[pl.ds(boff, 128)])
            return 0
        lax.fori_loop(0, nb, body, 0)

    return k(xw_x, map_flat, dinv_flat, hop_flat)


def _edge_kernel(y176, src_flat, dst_flat):
    @functools.partial(
        pl.kernel, mesh=_mesh(), compiler_params=_SC_PARAMS,
        out_type=jax.ShapeDtypeStruct((M, DW), F32),
        scratch_types=[
            pltpu.VMEM((EBLK,), I32),
            pltpu.VMEM((EBLK,), I32),
            pltpu.VMEM((2304,), I32),
            pltpu.VMEM((2304,), I32),
            pltpu.VMEM((1, 128), I32),
            pltpu.VMEM((1, 128), I32),
            pltpu.VMEM((1, 128), I32),
            pltpu.VMEM((1, 128), I32),
            pltpu.VMEM((128, DW), F32),
            pltpu.VMEM((128, DW), F32),
            pltpu.VMEM_SHARED((R + 16, DW), F32),
            pltpu.SemaphoreType.DMA,
            pltpu.SemaphoreType.DMA,
            pltpu.SemaphoreType.DMA,
        ],
    )
    def k(y_hbm, src_hbm, dst_hbm, agg_hbm, st_s, st_d, bsrc, bdst, isrc0,
          idst0, isrc1, idst1, rows0, rows1, acc, sem_g, sem_s0, sem_s1):
        c = lax.axis_index("c")
        s = lax.axis_index("s")
        trash = R + s

        def do_flush(off, fl):
            def one(isrc, idst, rows, sem_s):
                @pl.when(fl >= 2)
                def _():
                    pltpu.make_async_copy(rows, acc.at[idst.at[0]],
                                          sem_s).wait()

                def pack(kk, _):
                    isrc[0, pl.ds(kk * 16, 16)] = bsrc[
                        pl.ds(off + kk * 16, 16)]
                    idst[0, pl.ds(kk * 16, 16)] = bdst[
                        pl.ds(off + kk * 16, 16)]
                    return 0
                lax.fori_loop(0, 8, pack, 0)
                pltpu.async_copy(y_hbm.at[isrc.at[0]], rows, sem_g).wait()
                pltpu.async_copy(rows, acc.at[idst.at[0]], sem_s, add=True)

            @pl.when(lax.bitwise_and(fl, 1) == 0)
            def _():
                one(isrc0, idst0, rows0, sem_s0)

            @pl.when(lax.bitwise_and(fl, 1) == 1)
            def _():
                one(isrc1, idst1, rows1, sem_s1)

        def drain(fl):
            @pl.when(fl >= 1)
            def _():
                p = lax.bitwise_and(fl - 1, 1)

                @pl.when(p == 0)
                def _():
                    pltpu.make_async_copy(rows0, acc.at[idst0.at[0]],
                                          sem_s0).wait()

                @pl.when(p == 1)
                def _():
                    pltpu.make_async_copy(rows1, acc.at[idst1.at[0]],
                                          sem_s1).wait()

            @pl.when(fl >= 2)
            def _():
                p = lax.bitwise_and(fl, 1)

                @pl.when(p == 0)
                def _():
                    pltpu.make_async_copy(rows0, acc.at[idst0.at[0]],
                                          sem_s0).wait()

                @pl.when(p == 1)
                def _():
                    pltpu.make_async_copy(rows1, acc.at[idst1.at[0]],
                                          sem_s1).wait()

        def chunk(ci, _):
            lo = (c * 13 + ci) * R
            pltpu.sync_copy(y_hbm.at[pl.ds(lo + s * 400, 400)],
                            acc.at[pl.ds(s * 400, 400)])
            plsc.subcore_barrier()

            def block(b, carry):
                cnt, fl = carry
                base = s * STRIPE + b * EBLK
                pltpu.sync_copy(src_hbm.at[pl.ds(base, EBLK)], st_s)
                pltpu.sync_copy(dst_hbm.at[pl.ds(base, EBLK)], st_d)

                def fbody(t, cnt):
                    dv = st_d[pl.ds(t * 16, 16)]
                    sv = st_s[pl.ds(t * 16, 16)]
                    m = (dv >= lo) & (dv < lo + R)
                    plsc.store_compressed(bsrc.at[pl.ds(cnt, 16)], sv,
                                          mask=m)
                    plsc.store_compressed(bdst.at[pl.ds(cnt, 16)], dv - lo,
                                          mask=m)
                    return cnt + plsc.all_reduce_population_count(m)[0]
                cnt = lax.fori_loop(0, EBLK // 16, fbody, cnt)

                def flush(f, fl):
                    do_flush(f * 128, fl)
                    return fl + 1
                nf = lax.shift_right_logical(cnt, 7)
                fl = lax.fori_loop(0, nf, flush, fl)
                rem = lax.bitwise_and(cnt, 127)

                def mv(kk, _):
                    bsrc[pl.ds(kk * 16, 16)] = bsrc[
                        pl.ds(nf * 128 + kk * 16, 16)]
                    bdst[pl.ds(kk * 16, 16)] = bdst[
                        pl.ds(nf * 128 + kk * 16, 16)]
                    return 0
                lax.fori_loop(0, 8, mv, 0)
                return (rem, fl)
            cnt, fl = lax.fori_loop(0, STRIPE // EBLK, block,
                                    (jnp.int32(0), jnp.int32(0)))

            def pad(kk, _):
                bsrc[pl.ds(cnt + kk * 16, 16)] = jnp.zeros((16,), I32)
                bdst[pl.ds(cnt + kk * 16, 16)] = (
                    jnp.zeros((16,), I32) + trash)
                return 0
            lax.fori_loop(0, 8, pad, 0)

            @pl.when(cnt > 0)
            def _():
                do_flush(0, fl)

            fl = fl + jnp.where(cnt > 0, 1, 0)
            drain(fl)
            plsc.subcore_barrier()
            pltpu.sync_copy(acc.at[pl.ds(s * 400, 400)],
                            agg_hbm.at[pl.ds(lo + s * 400, 400)])
            plsc.subcore_barrier()
            return 0
        nchunks = 13 - c
        lax.fori_loop(0, nchunks, chunk, 0)

    return k(y176, src_flat, dst_flat)


def _ctx_kernel(ctxall, map3):
    @functools.partial(
        pl.kernel, mesh=_mesh(), compiler_params=_SC_PARAMS_TILED,
        out_type=jax.ShapeDtypeStruct((2, N, D), F32),
        scratch_types=[
            pltpu.VMEM((1, 128), I32),
            pltpu.VMEM((128, D), F32),
            pltpu.VMEM_SHARED((N, D), F32),
        ],
    )
    def k(ctxall_hbm, map_hbm, ctxp_hbm, irow, vals, acc):
        c = lax.axis_index("c")
        s = lax.axis_index("s")
        w = s * 2 + c
        def zv(kk, _):
            vals[kk, pl.ds(0, 16)] = jnp.zeros((16,), F32)
            vals[kk, pl.ds(16, 16)] = jnp.zeros((16,), F32)
            vals[kk, pl.ds(32, 16)] = jnp.zeros((16,), F32)
            vals[kk, pl.ds(48, 16)] = jnp.zeros((16,), F32)
            vals[kk, pl.ds(64, 16)] = jnp.zeros((16,), F32)
            vals[kk, pl.ds(80, 16)] = jnp.zeros((16,), F32)
            vals[kk, pl.ds(96, 16)] = jnp.zeros((16,), F32)
            vals[kk, pl.ds(112, 16)] = jnp.zeros((16,), F32)
            return 0
        lax.fori_loop(0, 128, zv, 0)

        def za(kk, _):
            pltpu.sync_copy(vals, acc.at[pl.ds(s * 624 + kk * 128, 128)])
            return 0
        lax.fori_loop(0, 4, za, 0)
        pltpu.sync_copy(vals.at[pl.ds(0, 112)],
                        acc.at[pl.ds(s * 624 + 512, 112)])

        @pl.when(s == 15)
        def _():
            pltpu.sync_copy(vals.at[pl.ds(0, 16)], acc.at[pl.ds(9984, 16)])

        plsc.subcore_barrier()
        nb = 39 + jnp.where(w < 2, 1, 0)

        def body(i, _):
            bb = w + 32 * i
            pltpu.sync_copy(map_hbm.at[bb], irow)
            pltpu.sync_copy(ctxall_hbm.at[pl.ds(bb * 128, 128)], vals)
            pltpu.sync_copy(vals, acc.at[irow.at[0]], add=True)
            return 0
        lax.fori_loop(0, nb, body, 0)
        plsc.subcore_barrier()
        pltpu.sync_copy(acc.at[pl.ds(s * 624, 624)],
                        ctxp_hbm.at[c, pl.ds(s * 624, 624)])

        @pl.when(s == 15)
        def _():
            pltpu.sync_copy(acc.at[pl.ds(9984, 16)],
                            ctxp_hbm.at[c, pl.ds(9984, 16)])

    return k(ctxall, map3)


def kernel(x, hop_table, gcn_W, gcn_b, bn_gamma, bn_beta, enc_W1, enc_b1,
           enc_W2, enc_b2, sub_W, sub_b, ctx_W, ctx_b, gc_W, gc_b, gs_W,
           gs_b, gx_W, gx_b, subgraphs_nodes_mapper, combined_subgraphs,
           subgraphs_batch, hop_indicator):
    del gcn_b
    del subgraphs_batch

    mapper = subgraphs_nodes_mapper.astype(I32)
    hop = hop_indicator.astype(I32)
    src = combined_subgraphs[0].astype(I32)
    dst = combined_subgraphs[1].astype(I32)

    pad = EPAD - E
    src_flat = jnp.concatenate([src, jnp.zeros((pad,), I32)])
    dst_flat = jnp.concatenate([dst, jnp.full((pad,), M, I32)])
    map3 = mapper.reshape(M // 128, 1, 128)
    hop_col = hop.reshape(M, 1)
    ones16 = jnp.ones((128, 16), F32)
    zeros16 = jnp.zeros((5000, 16), F32)
    row2 = lambda v: v.reshape(1, -1)

    xw_x, hopW32, gcT, gsT, gxT = pl.pallas_call(
        _tables_body,
        out_shape=(
            jax.ShapeDtypeStruct((N, DH), F32),
            jax.ShapeDtypeStruct((32, DH), F32),
            jax.ShapeDtypeStruct((20, D), F32),
            jax.ShapeDtypeStruct((20, D), F32),
            jax.ShapeDtypeStruct((20, D), F32),
        ),
    )(x, hop_table, gcn_W, gc_W, row2(gc_b), gs_W, row2(gs_b), gx_W,
      row2(gx_b))

    deg0, deg1, cnt0, cnt1 = _deg_cnt_kernel(dst_flat, mapper, ones16,
                                             zeros16)
    degcat = jnp.concatenate([deg0, deg1], axis=0)

    dinv = pl.pallas_call(
        _dinv_body,
        grid=(GRID_M,),
        in_specs=[pl.BlockSpec((MB, 16), lambda i: (i, 0))],
        out_specs=pl.BlockSpec((MB, 1), lambda i: (i, 0)),
        out_shape=jax.ShapeDtypeStruct((M, 1), F32),
    )(degcat)

    y176 = _ya_kernel(xw_x, mapper, dinv.reshape(M), hop)

    agg = _edge_kernel(y176, src_flat, dst_flat)

    stats = pl.pallas_call(
        _stats_body,
        grid=(GRID_M,),
        in_specs=[
            pl.BlockSpec((MB, DW), lambda i: (i, 0)),
            pl.BlockSpec((MB, 1), lambda i: (i, 0)),
            pl.BlockSpec((32, DH), lambda i: (0, 0)),
        ],
        out_specs=pl.BlockSpec((8, DH), lambda i: (0, 0)),
        out_shape=jax.ShapeDtypeStruct((8, DH), F32),
    )(agg, dinv, hopW32)

    ctxall, outp = pl.pallas_call(
        _main_body,
        grid=(GRID_M,),
        in_specs=[
            pl.BlockSpec((MB, DW), lambda i: (i, 0)),
            pl.BlockSpec((MB, 1), lambda i: (i, 0)),
            pl.BlockSpec((MB, 1), lambda i: (i, 0)),
            pl.BlockSpec((8, DH), lambda i: (0, 0)),
            pl.BlockSpec((32, DH), lambda i: (0, 0)),
            pl.BlockSpec((1, DH), lambda i: (0, 0)),
            pl.BlockSpec((1, DH), lambda i: (0, 0)),
            pl.BlockSpec((DH, D), lambda i: (0, 0)),
            pl.BlockSpec((1, D), lambda i: (0, 0)),
            pl.BlockSpec((D, D), lambda i: (0, 0)),
            pl.BlockSpec((1, D), lambda i: (0, 0)),
            pl.BlockSpec((D, D), lambda i: (0, 0)),
            pl.BlockSpec((1, D), lambda i: (0, 0)),
            pl.BlockSpec((D, D), lambda i: (0, 0)),
            pl.BlockSpec((1, D), lambda i: (0, 0)),
            pl.BlockSpec((20, D), lambda i: (0, 0)),
            pl.BlockSpec((20, D), lambda i: (0, 0)),
            pl.BlockSpec((20, D), lambda i: (0, 0)),
        ],
        out_specs=[
            pl.BlockSpec((MB, D), lambda i: (i, 0)),
            pl.BlockSpec((SUBB, D), lambda i: (i, 0)),
        ],
        out_shape=(
            jax.ShapeDtypeStruct((M, D), F32),
            jax.ShapeDtypeStruct((N, D), F32),
        ),
    )(agg, dinv, hop_col, stats, hopW32, row2(bn_gamma),
      row2(bn_beta), enc_W1, row2(enc_b1), enc_W2, row2(enc_b2), sub_W,
      row2(sub_b), ctx_W, row2(ctx_b), gcT, gsT, gxT)

    ctxp = _ctx_kernel(ctxall, map3)

    out = pl.pallas_call(
        _final_body,
        grid=(10,),
        in_specs=[
            pl.BlockSpec((1000, D), lambda i: (i, 0)),
            pl.BlockSpec((2, 1000, D), lambda i: (0, i, 0)),
            pl.BlockSpec((1000, 16), lambda i: (i, 0)),
            pl.BlockSpec((1000, 16), lambda i: (i, 0)),
        ],
        out_specs=pl.BlockSpec((1000, D), lambda i: (i, 0)),
        out_shape=jax.ShapeDtypeStruct((N, D), F32),
    )(outp, ctxp, cnt0, cnt1)

    return out


# speedup vs baseline: 1.1712x; 1.1712x over previous
import functools

import jax
import jax.numpy as jnp
from jax import lax
from jax.experimental import pallas as pl
from jax.experimental.pallas import tpu as pltpu
from jax.experimental.pallas import tpu_sc as plsc

F32 = jnp.float32
I32 = jnp.int32

N = 10000
S = 16
M = N * S
E = 320000
D = 128
H = 16
DH = D + H

EPAD = 327680
STRIPE = EPAD // 16
EBLK = 2048
MB = 1280
GRID_M = M // MB
SUBB = MB // S
DW = DH + 32
NCHUNK = 20
R = M // NCHUNK
HM = M // 2

_SC_PARAMS = pltpu.CompilerParams(
    needs_layout_passes=False, use_tc_tiling_on_sc=False)
_SC_PARAMS_TILED = pltpu.CompilerParams(needs_layout_passes=False)


def _mesh():
    return plsc.VectorSubcoreMesh(core_axis_name="c", subcore_axis_name="s")


def _tables_body(x_ref, ht_ref, w_ref, gcw_ref, gcb_ref, gsw_ref, gsb_ref,
                 gxw_ref, gxb_ref, xw_ref, hopw_ref, gct_ref, gst_ref,
                 gxt_ref):
    w = w_ref[...]
    xw_ref[...] = jnp.dot(x_ref[...], w[0:D, :], preferred_element_type=F32)
    ht = ht_ref[...]
    hw = jnp.dot(ht, w[D:DH, :], preferred_element_type=F32)
    hopw_ref[...] = jnp.concatenate([hw, jnp.zeros((12, DH), F32)], axis=0)
    gct_ref[...] = jax.nn.sigmoid(
        jnp.dot(ht, gcw_ref[...], preferred_element_type=F32) + gcb_ref[...])
    gst_ref[...] = jax.nn.sigmoid(
        jnp.dot(ht, gsw_ref[...], preferred_element_type=F32) + gsb_ref[...])
    gxt_ref[...] = jax.nn.sigmoid(
        jnp.dot(ht, gxw_ref[...], preferred_element_type=F32) + gxb_ref[...])


def _dinv_body(deg_ref, dinv_ref):
    dinv_ref[...] = lax.rsqrt(deg_ref[...][:, 0:1] + 1.0)


def _stats_body(agg_ref, dinv_ref, hopw_ref, out_ref):
    @pl.when(pl.program_id(0) == 0)
    def _():
        out_ref[...] = jnp.zeros_like(out_ref)
    agg = agg_ref[...]
    hb = dinv_ref[...] * (
        agg[:, 0:DH]
        + jnp.dot(agg[:, DH:DW], hopw_ref[...], preferred_element_type=F32))
    out_ref[0:1, :] += jnp.sum(hb, axis=0, keepdims=True)
    out_ref[1:2, :] += jnp.sum(hb * hb, axis=0, keepdims=True)


def _main_body(agg_ref, dinv_ref, hop_ref, stats_ref, hopw_ref,
               gamma_ref, beta_ref, w1_ref, b1_ref, w2_ref, b2_ref, subw_ref,
               subb_ref, ctxw_ref, ctxb_ref, gct_ref, gst_ref, gxt_ref,
               ctxall_ref, outp_ref):
    agg = agg_ref[...]
    hb = dinv_ref[...] * (
        agg[:, 0:DH]
        + jnp.dot(agg[:, DH:DW], hopw_ref[...], preferred_element_type=F32))
    stats = stats_ref[...]
    mu = stats[0:1, :] * (1.0 / M)
    var = stats[1:2, :] * (1.0 / M) - mu * mu
    xn = (hb - mu) * (gamma_ref[...] * lax.rsqrt(var + 1e-5)) + beta_ref[...]
    xn = jnp.maximum(xn, 0.0)
    t = jnp.maximum(jnp.dot(xn, w1_ref[...], preferred_element_type=F32)
                    + b1_ref[...], 0.0)
    h = jnp.dot(t, w2_ref[...], preferred_element_type=F32) + b2_ref[...]
    hop = hop_ref[...] + 1
    oh = (hop == lax.broadcasted_iota(I32, (MB, 20), 1)).astype(F32)
    gs = jnp.dot(oh, gst_ref[...], preferred_element_type=F32)
    gx = jnp.dot(oh, gxt_ref[...], preferred_element_type=F32)
    suball = (jnp.dot(h, subw_ref[...], preferred_element_type=F32)
              + subb_ref[...]) * gs
    ctxall_ref[...] = (jnp.dot(h, ctxw_ref[...], preferred_element_type=F32)
                       + ctxb_ref[...]) * gx
    r = lax.broadcasted_iota(I32, (SUBB, MB), 1)
    i = lax.broadcasted_iota(I32, (SUBB, MB), 0)
    pool = ((r // S) == i).astype(F32)
    subm = jnp.dot(pool, suball, preferred_element_type=F32) * (1.0 / S)
    csel = (r == i * S).astype(F32)
    hc = jnp.dot(csel, h, preferred_element_type=F32)
    ohc = jnp.dot(csel, oh, preferred_element_type=F32)
    gc = jnp.dot(ohc, gct_ref[...], preferred_element_type=F32)
    outp_ref[...] = subm + hc * gc


def _final_body(outp_ref, ctxp_ref, cnt0_ref, cnt1_ref, out_ref):
    ctxp = ctxp_ref[...]
    cnt = cnt0_ref[...][:, 0:1] + cnt1_ref[...][:, 0:1]
    out_ref[...] = outp_ref[...] + (ctxp[0] + ctxp[1]) / jnp.maximum(cnt, 1.0)


def _deg_cnt_kernel(dst_flat, map_flat, ones16, zeros16):
    @functools.partial(
        pl.kernel, mesh=_mesh(), compiler_params=_SC_PARAMS,
        out_type=(jax.ShapeDtypeStruct((HM, 16), F32),
                  jax.ShapeDtypeStruct((HM, 16), F32),
                  jax.ShapeDtypeStruct((N, 16), F32),
                  jax.ShapeDtypeStruct((N, 16), F32)),
        scratch_types=[
            pltpu.VMEM((EBLK,), I32),
            pltpu.VMEM((2304,), I32),
            pltpu.VMEM((1, 128), I32),
            pltpu.VMEM((128, 16), F32),
            pltpu.VMEM_SHARED((HM + 16, 16), F32),
        ],
    )
    def k(dst_hbm, map_hbm, ones_hbm, zero_hbm, d0_hbm, d1_hbm, c0_hbm,
          c1_hbm, st_d, bdst, idst, ones_v, accum):
        c = lax.axis_index("c")
        s = lax.axis_index("s")
        lo = c * HM
        trash = HM + lax.rem(s, 8)
        pltpu.sync_copy(ones_hbm, ones_v)
        pltpu.sync_copy(zero_hbm, accum.at[pl.ds(s * 5000, 5000)])

        @pl.when(s == 0)
        def _():
            pltpu.sync_copy(zero_hbm.at[pl.ds(0, 16)],
                            accum.at[pl.ds(HM, 16)])

        plsc.subcore_barrier()

        def block(b, cnt):
            pltpu.sync_copy(
                dst_hbm.at[pl.ds(s * STRIPE + b * EBLK, EBLK)], st_d)

            def fbody(t, cnt):
                dv = st_d[pl.ds(t * 16, 16)]
                m = (dv >= lo) & (dv < lo + HM)
                plsc.store_compressed(bdst.at[pl.ds(cnt, 16)], dv - lo,
                                      mask=m)
                return cnt + plsc.all_reduce_population_count(m)[0]
            cnt = lax.fori_loop(0, EBLK // 16, fbody, cnt)

            def flush(f, _):
                def pack(kk, _):
                    idst[0, pl.ds(kk * 16, 16)] = bdst[
                        pl.ds(f * 128 + kk * 16, 16)]
                    return 0
                lax.fori_loop(0, 8, pack, 0)
                pltpu.sync_copy(ones_v, accum.at[idst.at[0]], add=True)
                return 0
            nf = lax.shift_right_logical(cnt, 7)
            lax.fori_loop(0, nf, flush, 0)
            rem = lax.bitwise_and(cnt, 127)

            def mv(kk, _):
                bdst[pl.ds(kk * 16, 16)] = bdst[pl.ds(nf * 128 + kk * 16, 16)]
                return 0
            lax.fori_loop(0, 8, mv, 0)
            return rem
        cnt = lax.fori_loop(0, STRIPE // EBLK, block, jnp.int32(0))

        def pad(kk, _):
            bdst[pl.ds(cnt + kk * 16, 16)] = jnp.zeros((16,), I32) + trash
            return 0
        lax.fori_loop(0, 8, pad, 0)

        def pack2(kk, _):
            idst[0, pl.ds(kk * 16, 16)] = bdst[pl.ds(kk * 16, 16)]
            return 0

        @pl.when(cnt > 0)
        def _():
            lax.fori_loop(0, 8, pack2, 0)
            pltpu.sync_copy(ones_v, accum.at[idst.at[0]], add=True)

        plsc.subcore_barrier()

        @pl.when(c == 0)
        def _():
            pltpu.sync_copy(accum.at[pl.ds(s * 5000, 5000)],
                            d0_hbm.at[pl.ds(s * 5000, 5000)])

        @pl.when(c == 1)
        def _():
            pltpu.sync_copy(accum.at[pl.ds(s * 5000, 5000)],
                            d1_hbm.at[pl.ds(s * 5000, 5000)])

        plsc.subcore_barrier()

        def zc(kk, _):
            pltpu.sync_copy(zero_hbm.at[pl.ds(0, 625)],
                            accum.at[pl.ds(s * 625, 625)])
            return 0
        lax.fori_loop(0, 1, zc, 0)
        plsc.subcore_barrier()
        nbm = 39 + jnp.where(s == 0, 1, 0)
        bbase = s * 39 + jnp.where(s > 0, 1, 0)

        def mblock(i, _):
            boff = c * HM + (bbase + i) * 128
            pltpu.sync_copy(map_hbm.at[pl.ds(boff, 128)], idst.at[0])
            pltpu.sync_copy(ones_v, accum.at[idst.at[0]], add=True)
            return 0
        lax.fori_loop(0, nbm, mblock, 0)
        plsc.subcore_barrier()

        @pl.when(c == 0)
        def _():
            pltpu.sync_copy(accum.at[pl.ds(s * 625, 625)],
                            c0_hbm.at[pl.ds(s * 625, 625)])

        @pl.when(c == 1)
        def _():
            pltpu.sync_copy(accum.at[pl.ds(s * 625, 625)],
                            c1_hbm.at[pl.ds(s * 625, 625)])

    return k(dst_flat, map_flat, ones16, zeros16)


def _ya_kernel(xw_x, map_flat, dinv_flat, hop_flat):
    @functools.partial(
        pl.kernel, mesh=_mesh(), compiler_params=_SC_PARAMS,
        out_type=jax.ShapeDtypeStruct((M, DW), F32),
        scratch_types=[
            pltpu.VMEM((1, 128), I32),
            pltpu.VMEM((128,), F32),
            pltpu.VMEM((128,), I32),
            pltpu.VMEM((128, DH), F32),
            pltpu.VMEM((128, DW), F32),
            pltpu.SemaphoreType.DMA,
        ],
    )
    def k(xw_hbm, map_hbm, dinv_hbm, hop_hbm, y_hbm, irow, dstage, hstage,
          rows_v, ybuf, sem):
        c = lax.axis_index("c")
        s = lax.axis_index("s")
        w = s * 2 + c
        nb = 39 + jnp.where(w < 2, 1, 0)
        lane = lax.iota(I32, 16)

        def body(i, _):
            boff = (w + 32 * i) * 128
            pltpu.sync_copy(map_hbm.at[pl.ds(boff, 128)], irow.at[0])
            pltpu.async_copy(xw_hbm.at[irow.at[0]], rows_v, sem).wait()
            pltpu.sync_copy(dinv_hbm.at[pl.ds(boff, 128)], dstage)
            pltpu.sync_copy(hop_hbm.at[pl.ds(boff, 128)], hstage)

            def rbody(g, _):
                dvec = dstage[pl.ds(g * 16, 16)]
                hvec = hstage[pl.ds(g * 16, 16)] + 1
                for r0 in range(16):
                    r = g * 16 + r0
                    d = dvec[r0]
                    h = hvec[r0]
                    for kk in range(DH // 16):
                        ybuf[r, pl.ds(kk * 16, 16)] = (
                            rows_v[r, pl.ds(kk * 16, 16)] * d)
                    ybuf[r, pl.ds(DH, 16)] = jnp.where(lane == h, d, 0.0)
                    ybuf[r, pl.ds(DH + 16, 16)] = jnp.where(
                        lane == h - 16, d, 0.0)
                return 0
            lax.fori_loop(0, 8, rbody, 0)
            pltpu.sync_copy(ybuf, y_hbm.at[pl.ds(boff, 128)])
            return 0
        lax.fori_loop(0, nb, body, 0)

    return k(xw_x, map_flat, dinv_flat, hop_flat)


def _edge_kernel(y176, src_flat, dst_flat):
    @functools.partial(
        pl.kernel, mesh=_mesh(), compiler_params=_SC_PARAMS,
        out_type=jax.ShapeDtypeStruct((M, DW), F32),
        scratch_types=[
            pltpu.VMEM((EBLK,), I32),
            pltpu.VMEM((EBLK,), I32),
            pltpu.VMEM((2304,), I32),
            pltpu.VMEM((1, 128), I32),
            pltpu.VMEM((1, 128), I32),
            pltpu.VMEM((128, DW), F32),
            pltpu.VMEM_SHARED((R + 16, DW), F32),
            pltpu.SemaphoreType.DMA,
        ],
    )
    def k(y_hbm, src_hbm, dst_hbm, agg_hbm, st_s, st_d, bpk, isrc, idst,
          rows_v, acc, sem):
        c = lax.axis_index("c")
        s = lax.axis_index("s")
        trash = R + s

        def do_flush(off):
            def pack(kk, _):
                pv = bpk[pl.ds(off + kk * 16, 16)]
                isrc[0, pl.ds(kk * 16, 16)] = lax.bitwise_and(pv, 0x3FFFF)
                idst[0, pl.ds(kk * 16, 16)] = lax.shift_right_logical(pv, 18)
                return 0
            lax.fori_loop(0, 8, pack, 0)
            pltpu.async_copy(y_hbm.at[isrc.at[0]], rows_v, sem).wait()
            pltpu.sync_copy(rows_v, acc.at[idst.at[0]], add=True)

        def chunk(ci, _):
            lo = (c * (NCHUNK // 2) + ci) * R
            pltpu.sync_copy(y_hbm.at[pl.ds(lo + s * 500, 500)],
                            acc.at[pl.ds(s * 500, 500)])
            plsc.subcore_barrier()

            def block(b, cnt):
                base = s * STRIPE + b * EBLK
                pltpu.sync_copy(src_hbm.at[pl.ds(base, EBLK)], st_s)
                pltpu.sync_copy(dst_hbm.at[pl.ds(base, EBLK)], st_d)

                def fbody(t, cnt):
                    dv = st_d[pl.ds(t * 16, 16)]
                    sv = st_s[pl.ds(t * 16, 16)]
                    dl = dv - lo
                    m = dl.astype(jnp.uint32) < jnp.uint32(R)
                    pv = lax.bitwise_or(sv, lax.shift_left(dl, 18))
                    plsc.store_compressed(bpk.at[pl.ds(cnt, 16)], pv,
                                          mask=m)
                    return cnt + plsc.all_reduce_population_count(m)[0]
                cnt = lax.fori_loop(0, EBLK // 16, fbody, cnt)

                def flush(f, _):
                    do_flush(f * 128)
                    return 0
                nf = lax.shift_right_logical(cnt, 7)
                lax.fori_loop(0, nf, flush, 0)
                rem = lax.bitwise_and(cnt, 127)

                def mv(kk, _):
                    bpk[pl.ds(kk * 16, 16)] = bpk[
                        pl.ds(nf * 128 + kk * 16, 16)]
                    return 0
                lax.fori_loop(0, 8, mv, 0)
                return rem
            cnt = lax.fori_loop(0, STRIPE // EBLK, block, jnp.int32(0))

            def pad(kk, _):
                bpk[pl.ds(cnt + kk * 16, 16)] = (
                    jnp.zeros((16,), I32) + lax.shift_left(trash, 18))
                return 0
            lax.fori_loop(0, 8, pad, 0)

            @pl.when(cnt > 0)
            def _():
                do_flush(0)

            plsc.subcore_barrier()
            pltpu.sync_copy(acc.at[pl.ds(s * 500, 500)],
                            agg_hbm.at[pl.ds(lo + s * 500, 500)])
            plsc.subcore_barrier()
            return 0
        lax.fori_loop(0, NCHUNK // 2, chunk, 0)

    return k(y176, src_flat, dst_flat)


def _ctx_kernel(ctxall, map3):
    @functools.partial(
        pl.kernel, mesh=_mesh(), compiler_params=_SC_PARAMS_TILED,
        out_type=jax.ShapeDtypeStruct((2, N, D), F32),
        scratch_types=[
            pltpu.VMEM((1, 128), I32),
            pltpu.VMEM((128, D), F32),
            pltpu.VMEM_SHARED((N, D), F32),
        ],
    )
    def k(ctxall_hbm, map_hbm, ctxp_hbm, irow, vals, acc):
        c = lax.axis_index("c")
        s = lax.axis_index("s")
        w = s * 2 + c
        def zv(kk, _):
            vals[kk, pl.ds(0, 16)] = jnp.zeros((16,), F32)
            vals[kk, pl.ds(16, 16)] = jnp.zeros((16,), F32)
            vals[kk, pl.ds(32, 16)] = jnp.zeros((16,), F32)
            vals[kk, pl.ds(48, 16)] = jnp.zeros((16,), F32)
            vals[kk, pl.ds(64, 16)] = jnp.zeros((16,), F32)
            vals[kk, pl.ds(80, 16)] = jnp.zeros((16,), F32)
            vals[kk, pl.ds(96, 16)] = jnp.zeros((16,), F32)
            vals[kk, pl.ds(112, 16)] = jnp.zeros((16,), F32)
            return 0
        lax.fori_loop(0, 128, zv, 0)

        def za(kk, _):
            pltpu.sync_copy(vals, acc.at[pl.ds(s * 624 + kk * 128, 128)])
            return 0
        lax.fori_loop(0, 4, za, 0)
        pltpu.sync_copy(vals.at[pl.ds(0, 112)],
                        acc.at[pl.ds(s * 624 + 512, 112)])

        @pl.when(s == 15)
        def _():
            pltpu.sync_copy(vals.at[pl.ds(0, 16)], acc.at[pl.ds(9984, 16)])

        plsc.subcore_barrier()
        nb = 39 + jnp.where(w < 2, 1, 0)

        def body(i, _):
            bb = w + 32 * i
            pltpu.sync_copy(map_hbm.at[bb], irow)
            pltpu.sync_copy(ctxall_hbm.at[pl.ds(bb * 128, 128)], vals)
            pltpu.sync_copy(vals, acc.at[irow.at[0]], add=True)
            return 0
        lax.fori_loop(0, nb, body, 0)
        plsc.subcore_barrier()
        pltpu.sync_copy(acc.at[pl.ds(s * 624, 624)],
                        ctxp_hbm.at[c, pl.ds(s * 624, 624)])

        @pl.when(s == 15)
        def _():
            pltpu.sync_copy(acc.at[pl.ds(9984, 16)],
                            ctxp_hbm.at[c, pl.ds(9984, 16)])

    return k(ctxall, map3)


def kernel(x, hop_table, gcn_W, gcn_b, bn_gamma, bn_beta, enc_W1, enc_b1,
           enc_W2, enc_b2, sub_W, sub_b, ctx_W, ctx_b, gc_W, gc_b, gs_W,
           gs_b, gx_W, gx_b, subgraphs_nodes_mapper, combined_subgraphs,
           subgraphs_batch, hop_indicator):
    del gcn_b
    del subgraphs_batch

    mapper = subgraphs_nodes_mapper.astype(I32)
    hop = hop_indicator.astype(I32)
    src = combined_subgraphs[0].astype(I32)
    dst = combined_subgraphs[1].astype(I32)

    pad = EPAD - E
    src_flat = jnp.concatenate([src, jnp.zeros((pad,), I32)])
    dst_flat = jnp.concatenate([dst, jnp.full((pad,), M, I32)])
    map3 = mapper.reshape(M // 128, 1, 128)
    hop_col = hop.reshape(M, 1)
    ones16 = jnp.ones((128, 16), F32)
    zeros16 = jnp.zeros((5000, 16), F32)
    row2 = lambda v: v.reshape(1, -1)

    xw_x, hopW32, gcT, gsT, gxT = pl.pallas_call(
        _tables_body,
        out_shape=(
            jax.ShapeDtypeStruct((N, DH), F32),
            jax.ShapeDtypeStruct((32, DH), F32),
            jax.ShapeDtypeStruct((20, D), F32),
            jax.ShapeDtypeStruct((20, D), F32),
            jax.ShapeDtypeStruct((20, D), F32),
        ),
    )(x, hop_table, gcn_W, gc_W, row2(gc_b), gs_W, row2(gs_b), gx_W,
      row2(gx_b))

    deg0, deg1, cnt0, cnt1 = _deg_cnt_kernel(dst_flat, mapper, ones16,
                                             zeros16)
    degcat = jnp.concatenate([deg0, deg1], axis=0)

    dinv = pl.pallas_call(
        _dinv_body,
        grid=(GRID_M,),
        in_specs=[pl.BlockSpec((MB, 16), lambda i: (i, 0))],
        out_specs=pl.BlockSpec((MB, 1), lambda i: (i, 0)),
        out_shape=jax.ShapeDtypeStruct((M, 1), F32),
    )(degcat)

    y176 = _ya_kernel(xw_x, mapper, dinv.reshape(M), hop)

    agg = _edge_kernel(y176, src_flat, dst_flat)

    stats = pl.pallas_call(
        _stats_body,
        grid=(GRID_M,),
        in_specs=[
            pl.BlockSpec((MB, DW), lambda i: (i, 0)),
            pl.BlockSpec((MB, 1), lambda i: (i, 0)),
            pl.BlockSpec((32, DH), lambda i: (0, 0)),
        ],
        out_specs=pl.BlockSpec((8, DH), lambda i: (0, 0)),
        out_shape=jax.ShapeDtypeStruct((8, DH), F32),
    )(agg, dinv, hopW32)

    ctxall, outp = pl.pallas_call(
        _main_body,
        grid=(GRID_M,),
        in_specs=[
            pl.BlockSpec((MB, DW), lambda i: (i, 0)),
            pl.BlockSpec((MB, 1), lambda i: (i, 0)),
            pl.BlockSpec((MB, 1), lambda i: (i, 0)),
            pl.BlockSpec((8, DH), lambda i: (0, 0)),
            pl.BlockSpec((32, DH), lambda i: (0, 0)),
            pl.BlockSpec((1, DH), lambda i: (0, 0)),
            pl.BlockSpec((1, DH), lambda i: (0, 0)),
            pl.BlockSpec((DH, D), lambda i: (0, 0)),
            pl.BlockSpec((1, D), lambda i: (0, 0)),
            pl.BlockSpec((D, D), lambda i: (0, 0)),
            pl.BlockSpec((1, D), lambda i: (0, 0)),
            pl.BlockSpec((D, D), lambda i: (0, 0)),
            pl.BlockSpec((1, D), lambda i: (0, 0)),
            pl.BlockSpec((D, D), lambda i: (0, 0)),
            pl.BlockSpec((1, D), lambda i: (0, 0)),
            pl.BlockSpec((20, D), lambda i: (0, 0)),
            pl.BlockSpec((20, D), lambda i: (0, 0)),
            pl.BlockSpec((20, D), lambda i: (0, 0)),
        ],
        out_specs=[
            pl.BlockSpec((MB, D), lambda i: (i, 0)),
            pl.BlockSpec((SUBB, D), lambda i: (i, 0)),
        ],
        out_shape=(
            jax.ShapeDtypeStruct((M, D), F32),
            jax.ShapeDtypeStruct((N, D), F32),
        ),
    )(agg, dinv, hop_col, stats, hopW32, row2(bn_gamma),
      row2(bn_beta), enc_W1, row2(enc_b1), enc_W2, row2(enc_b2), sub_W,
      row2(sub_b), ctx_W, row2(ctx_b), gcT, gsT, gxT)

    ctxp = _ctx_kernel(ctxall, map3)

    out = pl.pallas_call(
        _final_body,
        grid=(10,),
        in_specs=[
            pl.BlockSpec((1000, D), lambda i: (i, 0)),
            pl.BlockSpec((2, 1000, D), lambda i: (0, i, 0)),
            pl.BlockSpec((1000, 16), lambda i: (i, 0)),
            pl.BlockSpec((1000, 16), lambda i: (i, 0)),
        ],
        out_specs=pl.BlockSpec((1000, D), lambda i: (i, 0)),
        out_shape=jax.ShapeDtypeStruct((N, D), F32),
    )(outp, ctxp, cnt0, cnt1)

    return out


# speedup vs baseline: 1.2930x; 1.1040x over previous
import functools

import jax
import jax.numpy as jnp
from jax import lax
from jax.experimental import pallas as pl
from jax.experimental.pallas import tpu as pltpu
from jax.experimental.pallas import tpu_sc as plsc

F32 = jnp.float32
I32 = jnp.int32

N = 10000
S = 16
M = N * S
E = 320000
D = 128
H = 16
DH = D + H

EPAD = 327680
STRIPE = EPAD // 16
EBLK = 2048
MB = 1280
GRID_M = M // MB
SUBB = MB // S
DW = DH + 32
NCHUNK = 20
R = M // NCHUNK
HM = M // 2

_SC_PARAMS = pltpu.CompilerParams(
    needs_layout_passes=False, use_tc_tiling_on_sc=False)
_SC_PARAMS_TILED = pltpu.CompilerParams(needs_layout_passes=False)


def _mesh():
    return plsc.VectorSubcoreMesh(core_axis_name="c", subcore_axis_name="s")


def _tables_body(x_ref, ht_ref, w_ref, gcw_ref, gcb_ref, gsw_ref, gsb_ref,
                 gxw_ref, gxb_ref, xw_ref, hopw_ref, gct_ref, gst_ref,
                 gxt_ref):
    w = w_ref[...]
    xw_ref[...] = jnp.dot(x_ref[...], w[0:D, :], preferred_element_type=F32)
    ht = ht_ref[...]
    hw = jnp.dot(ht, w[D:DH, :], preferred_element_type=F32)
    hopw_ref[...] = jnp.concatenate([hw, jnp.zeros((12, DH), F32)], axis=0)
    gct_ref[...] = jax.nn.sigmoid(
        jnp.dot(ht, gcw_ref[...], preferred_element_type=F32) + gcb_ref[...])
    gst_ref[...] = jax.nn.sigmoid(
        jnp.dot(ht, gsw_ref[...], preferred_element_type=F32) + gsb_ref[...])
    gxt_ref[...] = jax.nn.sigmoid(
        jnp.dot(ht, gxw_ref[...], preferred_element_type=F32) + gxb_ref[...])


def _dinv_body(deg_ref, dinv_ref):
    dinv_ref[...] = lax.rsqrt(deg_ref[...][:, 0:1] + 1.0)


def _stats_body(agg_ref, dinv_ref, hopw_ref, out_ref):
    @pl.when(pl.program_id(0) == 0)
    def _():
        out_ref[...] = jnp.zeros_like(out_ref)
    agg = agg_ref[...]
    hb = dinv_ref[...] * (
        agg[:, 0:DH]
        + jnp.dot(agg[:, DH:DW], hopw_ref[...], preferred_element_type=F32))
    out_ref[0:1, :] += jnp.sum(hb, axis=0, keepdims=True)
    out_ref[1:2, :] += jnp.sum(hb * hb, axis=0, keepdims=True)


def _main_body(agg_ref, dinv_ref, hop_ref, stats_ref, hopw_ref,
               gamma_ref, beta_ref, w1_ref, b1_ref, w2_ref, b2_ref, subw_ref,
               subb_ref, ctxw_ref, ctxb_ref, gct_ref, gst_ref, gxt_ref,
               ctxall_ref, outp_ref):
    agg = agg_ref[...]
    hb = dinv_ref[...] * (
        agg[:, 0:DH]
        + jnp.dot(agg[:, DH:DW], hopw_ref[...], preferred_element_type=F32))
    stats = stats_ref[...]
    mu = stats[0:1, :] * (1.0 / M)
    var = stats[1:2, :] * (1.0 / M) - mu * mu
    xn = (hb - mu) * (gamma_ref[...] * lax.rsqrt(var + 1e-5)) + beta_ref[...]
    xn = jnp.maximum(xn, 0.0)
    t = jnp.maximum(jnp.dot(xn, w1_ref[...], preferred_element_type=F32)
                    + b1_ref[...], 0.0)
    h = jnp.dot(t, w2_ref[...], preferred_element_type=F32) + b2_ref[...]
    hop = hop_ref[...] + 1
    oh = (hop == lax.broadcasted_iota(I32, (MB, 20), 1)).astype(F32)
    gs = jnp.dot(oh, gst_ref[...], preferred_element_type=F32)
    gx = jnp.dot(oh, gxt_ref[...], preferred_element_type=F32)
    suball = (jnp.dot(h, subw_ref[...], preferred_element_type=F32)
              + subb_ref[...]) * gs
    ctxall_ref[...] = (jnp.dot(h, ctxw_ref[...], preferred_element_type=F32)
                       + ctxb_ref[...]) * gx
    r = lax.broadcasted_iota(I32, (SUBB, MB), 1)
    i = lax.broadcasted_iota(I32, (SUBB, MB), 0)
    pool = ((r // S) == i).astype(F32)
    subm = jnp.dot(pool, suball, preferred_element_type=F32) * (1.0 / S)
    csel = (r == i * S).astype(F32)
    hc = jnp.dot(csel, h, preferred_element_type=F32)
    ohc = jnp.dot(csel, oh, preferred_element_type=F32)
    gc = jnp.dot(ohc, gct_ref[...], preferred_element_type=F32)
    outp_ref[...] = subm + hc * gc


def _final_body(outp_ref, ctxp_ref, cnt0_ref, cnt1_ref, out_ref):
    ctxp = ctxp_ref[...]
    cnt = cnt0_ref[...][:, 0:1] + cnt1_ref[...][:, 0:1]
    out_ref[...] = outp_ref[...] + (ctxp[0] + ctxp[1]) / jnp.maximum(cnt, 1.0)


def _deg_cnt_kernel(dst_flat, map_flat, ones16, zeros16):
    @functools.partial(
        pl.kernel, mesh=_mesh(), compiler_params=_SC_PARAMS,
        out_type=(jax.ShapeDtypeStruct((HM, 16), F32),
                  jax.ShapeDtypeStruct((HM, 16), F32),
                  jax.ShapeDtypeStruct((N, 16), F32),
                  jax.ShapeDtypeStruct((N, 16), F32)),
        scratch_types=[
            pltpu.VMEM((EBLK,), I32),
            pltpu.VMEM((2304,), I32),
            pltpu.VMEM((1, 128), I32),
            pltpu.VMEM((128, 16), F32),
            pltpu.VMEM_SHARED((HM + 16, 16), F32),
        ],
    )
    def k(dst_hbm, map_hbm, ones_hbm, zero_hbm, d0_hbm, d1_hbm, c0_hbm,
          c1_hbm, st_d, bdst, idst, ones_v, accum):
        c = lax.axis_index("c")
        s = lax.axis_index("s")
        lo = c * HM
        trash = HM + lax.rem(s, 8)
        pltpu.sync_copy(ones_hbm, ones_v)
        pltpu.sync_copy(zero_hbm, accum.at[pl.ds(s * 5000, 5000)])

        @pl.when(s == 0)
        def _():
            pltpu.sync_copy(zero_hbm.at[pl.ds(0, 16)],
                            accum.at[pl.ds(HM, 16)])

        plsc.subcore_barrier()

        def block(b, cnt):
            pltpu.sync_copy(
                dst_hbm.at[pl.ds(s * STRIPE + b * EBLK, EBLK)], st_d)

            def fbody(t, cnt):
                dv = st_d[pl.ds(t * 16, 16)]
                m = (dv >= lo) & (dv < lo + HM)
                plsc.store_compressed(bdst.at[pl.ds(cnt, 16)], dv - lo,
                                      mask=m)
                return cnt + plsc.all_reduce_population_count(m)[0]
            cnt = lax.fori_loop(0, EBLK // 16, fbody, cnt)

            def flush(f, _):
                def pack(kk, _):
                    idst[0, pl.ds(kk * 16, 16)] = bdst[
                        pl.ds(f * 128 + kk * 16, 16)]
                    return 0
                lax.fori_loop(0, 8, pack, 0)
                pltpu.sync_copy(ones_v, accum.at[idst.at[0]], add=True)
                return 0
            nf = lax.shift_right_logical(cnt, 7)
            lax.fori_loop(0, nf, flush, 0)
            rem = lax.bitwise_and(cnt, 127)

            def mv(kk, _):
                bdst[pl.ds(kk * 16, 16)] = bdst[pl.ds(nf * 128 + kk * 16, 16)]
                return 0
            lax.fori_loop(0, 8, mv, 0)
            return rem
        cnt = lax.fori_loop(0, STRIPE // EBLK, block, jnp.int32(0))

        def pad(kk, _):
            bdst[pl.ds(cnt + kk * 16, 16)] = jnp.zeros((16,), I32) + trash
            return 0
        lax.fori_loop(0, 8, pad, 0)

        def pack2(kk, _):
            idst[0, pl.ds(kk * 16, 16)] = bdst[pl.ds(kk * 16, 16)]
            return 0

        @pl.when(cnt > 0)
        def _():
            lax.fori_loop(0, 8, pack2, 0)
            pltpu.sync_copy(ones_v, accum.at[idst.at[0]], add=True)

        plsc.subcore_barrier()

        @pl.when(c == 0)
        def _():
            pltpu.sync_copy(accum.at[pl.ds(s * 5000, 5000)],
                            d0_hbm.at[pl.ds(s * 5000, 5000)])

        @pl.when(c == 1)
        def _():
            pltpu.sync_copy(accum.at[pl.ds(s * 5000, 5000)],
                            d1_hbm.at[pl.ds(s * 5000, 5000)])

        plsc.subcore_barrier()

        def zc(kk, _):
            pltpu.sync_copy(zero_hbm.at[pl.ds(0, 625)],
                            accum.at[pl.ds(s * 625, 625)])
            return 0
        lax.fori_loop(0, 1, zc, 0)
        plsc.subcore_barrier()
        nbm = 39 + jnp.where(s == 0, 1, 0)
        bbase = s * 39 + jnp.where(s > 0, 1, 0)

        def mblock(i, _):
            boff = c * HM + (bbase + i) * 128
            pltpu.sync_copy(map_hbm.at[pl.ds(boff, 128)], idst.at[0])
            pltpu.sync_copy(ones_v, accum.at[idst.at[0]], add=True)
            return 0
        lax.fori_loop(0, nbm, mblock, 0)
        plsc.subcore_barrier()

        @pl.when(c == 0)
        def _():
            pltpu.sync_copy(accum.at[pl.ds(s * 625, 625)],
                            c0_hbm.at[pl.ds(s * 625, 625)])

        @pl.when(c == 1)
        def _():
            pltpu.sync_copy(accum.at[pl.ds(s * 625, 625)],
                            c1_hbm.at[pl.ds(s * 625, 625)])

    return k(dst_flat, map_flat, ones16, zeros16)


def _ya_kernel(xw_x, map_flat, dinv_flat, hop_flat):
    @functools.partial(
        pl.kernel, mesh=_mesh(), compiler_params=_SC_PARAMS,
        out_type=jax.ShapeDtypeStruct((M, DW), F32),
        scratch_types=[
            pltpu.VMEM((1, 128), I32),
            pltpu.VMEM((128,), F32),
            pltpu.VMEM((128,), I32),
            pltpu.VMEM((128, DH), F32),
            pltpu.VMEM((128, 32), F32),
            pltpu.SemaphoreType.DMA,
        ],
    )
    def k(xw_hbm, map_hbm, dinv_hbm, hop_hbm, y_hbm, irow, dstage, hstage,
          rows_v, ohbuf, sem):
        c = lax.axis_index("c")
        s = lax.axis_index("s")
        w = s * 2 + c
        nb = 39 + jnp.where(w < 2, 1, 0)
        lane = lax.iota(I32, 16)

        def body(i, _):
            boff = (w + 32 * i) * 128
            pltpu.sync_copy(map_hbm.at[pl.ds(boff, 128)], irow.at[0])
            pltpu.async_copy(xw_hbm.at[irow.at[0]], rows_v, sem).wait()
            pltpu.sync_copy(dinv_hbm.at[pl.ds(boff, 128)], dstage)
            pltpu.sync_copy(hop_hbm.at[pl.ds(boff, 128)], hstage)

            def rbody(g, _):
                dvec = dstage[pl.ds(g * 16, 16)]
                hvec = hstage[pl.ds(g * 16, 16)] + 1
                for r0 in range(16):
                    r = g * 16 + r0
                    d = dvec[r0]
                    h = hvec[r0]
                    for kk in range(DH // 16):
                        rows_v[r, pl.ds(kk * 16, 16)] = (
                            rows_v[r, pl.ds(kk * 16, 16)] * d)
                    ohbuf[r, pl.ds(0, 16)] = jnp.where(lane == h, d, 0.0)
                    ohbuf[r, pl.ds(16, 16)] = jnp.where(lane == h - 16, d,
                                                        0.0)
                return 0
            lax.fori_loop(0, 8, rbody, 0)
            pltpu.sync_copy(rows_v,
                            y_hbm.at[pl.ds(boff, 128), pl.ds(0, DH)])
            pltpu.sync_copy(ohbuf,
                            y_hbm.at[pl.ds(boff, 128), pl.ds(DH, 32)])
            return 0
        lax.fori_loop(0, nb, body, 0)

    return k(xw_x, map_flat, dinv_flat, hop_flat)


def _edge_kernel(y176, src_flat, dst_flat):
    @functools.partial(
        pl.kernel, mesh=_mesh(), compiler_params=_SC_PARAMS,
        out_type=jax.ShapeDtypeStruct((M, DW), F32),
        scratch_types=[
            pltpu.VMEM((EBLK,), I32),
            pltpu.VMEM((EBLK,), I32),
            pltpu.VMEM((EBLK,), I32),
            pltpu.VMEM((EBLK,), I32),
            pltpu.VMEM((2304,), I32),
            pltpu.VMEM((1, 128), I32),
            pltpu.VMEM((1, 128), I32),
            pltpu.VMEM((128, DW), F32),
            pltpu.VMEM_SHARED((R + 16, DW), F32),
            pltpu.SemaphoreType.DMA,
            pltpu.SemaphoreType.DMA,
        ],
    )
    def k(y_hbm, src_hbm, dst_hbm, agg_hbm, st_s0, st_d0, st_s1, st_d1, bpk,
          isrc, idst, rows_v, acc, sem, semstg):
        c = lax.axis_index("c")
        s = lax.axis_index("s")
        trash = R + s

        def do_flush(off):
            def pack(kk, _):
                pv = bpk[pl.ds(off + kk * 16, 16)]
                isrc[0, pl.ds(kk * 16, 16)] = lax.bitwise_and(pv, 0x3FFFF)
                idst[0, pl.ds(kk * 16, 16)] = lax.shift_right_logical(pv, 18)
                return 0
            lax.fori_loop(0, 8, pack, 0)
            pltpu.async_copy(y_hbm.at[isrc.at[0]], rows_v, sem).wait()
            pltpu.sync_copy(rows_v, acc.at[idst.at[0]], add=True)

        def filter_block(st_s, st_d, lo, cnt):
            def fbody(t, cnt):
                dv = st_d[pl.ds(t * 16, 16)]
                sv = st_s[pl.ds(t * 16, 16)]
                dl = dv - lo
                m = dl.astype(jnp.uint32) < jnp.uint32(R)
                pv = lax.bitwise_or(sv, lax.shift_left(dl, 18))
                plsc.store_compressed(bpk.at[pl.ds(cnt, 16)], pv, mask=m)
                return cnt + plsc.all_reduce_population_count(m)[0]
            cnt = lax.fori_loop(0, EBLK // 16, fbody, cnt)

            def flush(f, _):
                do_flush(f * 128)
                return 0
            nf = lax.shift_right_logical(cnt, 7)
            lax.fori_loop(0, nf, flush, 0)
            rem = lax.bitwise_and(cnt, 127)

            def mv(kk, _):
                bpk[pl.ds(kk * 16, 16)] = bpk[pl.ds(nf * 128 + kk * 16, 16)]
                return 0
            lax.fori_loop(0, 8, mv, 0)
            return rem

        NBLK = STRIPE // EBLK

        def chunk(ci, _):
            lo = (c * (NCHUNK // 2) + ci) * R
            pltpu.sync_copy(y_hbm.at[pl.ds(lo + s * 500, 500)],
                            acc.at[pl.ds(s * 500, 500)])
            plsc.subcore_barrier()
            sbase = s * STRIPE
            pltpu.sync_copy(src_hbm.at[pl.ds(sbase, EBLK)], st_s0)
            pltpu.sync_copy(dst_hbm.at[pl.ds(sbase, EBLK)], st_d0)

            def dblock(bb, cnt):
                b0 = 2 * bb
                base1 = sbase + (b0 + 1) * EBLK
                d1 = pltpu.async_copy(src_hbm.at[pl.ds(base1, EBLK)], st_s1,
                                      semstg)
                d2 = pltpu.async_copy(dst_hbm.at[pl.ds(base1, EBLK)], st_d1,
                                      semstg)
                cnt = filter_block(st_s0, st_d0, lo, cnt)
                d1.wait()
                d2.wait()
                nxt = sbase + lax.rem(b0 + 2, NBLK) * EBLK

                @pl.when(bb < NBLK // 2 - 1)
                def _():
                    pltpu.async_copy(src_hbm.at[pl.ds(nxt, EBLK)], st_s0,
                                     semstg)
                    pltpu.async_copy(dst_hbm.at[pl.ds(nxt, EBLK)], st_d0,
                                     semstg)
                cnt = filter_block(st_s1, st_d1, lo, cnt)

                @pl.when(bb < NBLK // 2 - 1)
                def _():
                    pltpu.make_async_copy(
                        src_hbm.at[pl.ds(nxt, EBLK)], st_s0, semstg).wait()
                    pltpu.make_async_copy(
                        dst_hbm.at[pl.ds(nxt, EBLK)], st_d0, semstg).wait()
                return cnt
            cnt = lax.fori_loop(0, NBLK // 2, dblock, jnp.int32(0))

            def pad(kk, _):
                bpk[pl.ds(cnt + kk * 16, 16)] = (
                    jnp.zeros((16,), I32) + lax.shift_left(trash, 18))
                return 0
            lax.fori_loop(0, 8, pad, 0)

            @pl.when(cnt > 0)
            def _():
                do_flush(0)

            plsc.subcore_barrier()
            pltpu.sync_copy(acc.at[pl.ds(s * 500, 500)],
                            agg_hbm.at[pl.ds(lo + s * 500, 500)])
            plsc.subcore_barrier()
            return 0
        lax.fori_loop(0, NCHUNK // 2, chunk, 0)

    return k(y176, src_flat, dst_flat)


def _ctx_kernel(ctxall, map3):
    @functools.partial(
        pl.kernel, mesh=_mesh(), compiler_params=_SC_PARAMS_TILED,
        out_type=jax.ShapeDtypeStruct((2, N, D), F32),
        scratch_types=[
            pltpu.VMEM((1, 128), I32),
            pltpu.VMEM((128, D), F32),
            pltpu.VMEM_SHARED((N, D), F32),
        ],
    )
    def k(ctxall_hbm, map_hbm, ctxp_hbm, irow, vals, acc):
        c = lax.axis_index("c")
        s = lax.axis_index("s")
        w = s * 2 + c
        def zv(kk, _):
            vals[kk, pl.ds(0, 16)] = jnp.zeros((16,), F32)
            vals[kk, pl.ds(16, 16)] = jnp.zeros((16,), F32)
            vals[kk, pl.ds(32, 16)] = jnp.zeros((16,), F32)
            vals[kk, pl.ds(48, 16)] = jnp.zeros((16,), F32)
            vals[kk, pl.ds(64, 16)] = jnp.zeros((16,), F32)
            vals[kk, pl.ds(80, 16)] = jnp.zeros((16,), F32)
            vals[kk, pl.ds(96, 16)] = jnp.zeros((16,), F32)
            vals[kk, pl.ds(112, 16)] = jnp.zeros((16,), F32)
            return 0
        lax.fori_loop(0, 128, zv, 0)

        def za(kk, _):
            pltpu.sync_copy(vals, acc.at[pl.ds(s * 624 + kk * 128, 128)])
            return 0
        lax.fori_loop(0, 4, za, 0)
        pltpu.sync_copy(vals.at[pl.ds(0, 112)],
                        acc.at[pl.ds(s * 624 + 512, 112)])

        @pl.when(s == 15)
        def _():
            pltpu.sync_copy(vals.at[pl.ds(0, 16)], acc.at[pl.ds(9984, 16)])

        plsc.subcore_barrier()
        nb = 39 + jnp.where(w < 2, 1, 0)

        def body(i, _):
            bb = w + 32 * i
            pltpu.sync_copy(map_hbm.at[bb], irow)
            pltpu.sync_copy(ctxall_hbm.at[pl.ds(bb * 128, 128)], vals)
            pltpu.sync_copy(vals, acc.at[irow.at[0]], add=True)
            return 0
        lax.fori_loop(0, nb, body, 0)
        plsc.subcore_barrier()
        pltpu.sync_copy(acc.at[pl.ds(s * 624, 624)],
                        ctxp_hbm.at[c, pl.ds(s * 624, 624)])

        @pl.when(s == 15)
        def _():
            pltpu.sync_copy(acc.at[pl.ds(9984, 16)],
                            ctxp_hbm.at[c, pl.ds(9984, 16)])

    return k(ctxall, map3)


def kernel(x, hop_table, gcn_W, gcn_b, bn_gamma, bn_beta, enc_W1, enc_b1,
           enc_W2, enc_b2, sub_W, sub_b, ctx_W, ctx_b, gc_W, gc_b, gs_W,
           gs_b, gx_W, gx_b, subgraphs_nodes_mapper, combined_subgraphs,
           subgraphs_batch, hop_indicator):
    del gcn_b
    del subgraphs_batch

    mapper = subgraphs_nodes_mapper.astype(I32)
    hop = hop_indicator.astype(I32)
    src = combined_subgraphs[0].astype(I32)
    dst = combined_subgraphs[1].astype(I32)

    pad = EPAD - E
    src_flat = jnp.concatenate([src, jnp.zeros((pad,), I32)])
    dst_flat = jnp.concatenate([dst, jnp.full((pad,), M, I32)])
    map3 = mapper.reshape(M // 128, 1, 128)
    hop_col = hop.reshape(M, 1)
    ones16 = jnp.ones((128, 16), F32)
    zeros16 = jnp.zeros((5000, 16), F32)
    row2 = lambda v: v.reshape(1, -1)

    xw_x, hopW32, gcT, gsT, gxT = pl.pallas_call(
        _tables_body,
        out_shape=(
            jax.ShapeDtypeStruct((N, DH), F32),
            jax.ShapeDtypeStruct((32, DH), F32),
            jax.ShapeDtypeStruct((20, D), F32),
            jax.ShapeDtypeStruct((20, D), F32),
            jax.ShapeDtypeStruct((20, D), F32),
        ),
    )(x, hop_table, gcn_W, gc_W, row2(gc_b), gs_W, row2(gs_b), gx_W,
      row2(gx_b))

    deg0, deg1, cnt0, cnt1 = _deg_cnt_kernel(dst_flat, mapper, ones16,
                                             zeros16)
    degcat = jnp.concatenate([deg0, deg1], axis=0)

    dinv = pl.pallas_call(
        _dinv_body,
        grid=(GRID_M,),
        in_specs=[pl.BlockSpec((MB, 16), lambda i: (i, 0))],
        out_specs=pl.BlockSpec((MB, 1), lambda i: (i, 0)),
        out_shape=jax.ShapeDtypeStruct((M, 1), F32),
    )(degcat)

    y176 = _ya_kernel(xw_x, mapper, dinv.reshape(M), hop)

    agg = _edge_kernel(y176, src_flat, dst_flat)

    stats = pl.pallas_call(
        _stats_body,
        grid=(GRID_M,),
        in_specs=[
            pl.BlockSpec((MB, DW), lambda i: (i, 0)),
            pl.BlockSpec((MB, 1), lambda i: (i, 0)),
            pl.BlockSpec((32, DH), lambda i: (0, 0)),
        ],
        out_specs=pl.BlockSpec((8, DH), lambda i: (0, 0)),
        out_shape=jax.ShapeDtypeStruct((8, DH), F32),
    )(agg, dinv, hopW32)

    ctxall, outp = pl.pallas_call(
        _main_body,
        grid=(GRID_M,),
        in_specs=[
            pl.BlockSpec((MB, DW), lambda i: (i, 0)),
            pl.BlockSpec((MB, 1), lambda i: (i, 0)),
            pl.BlockSpec((MB, 1), lambda i: (i, 0)),
            pl.BlockSpec((8, DH), lambda i: (0, 0)),
            pl.BlockSpec((32, DH), lambda i: (0, 0)),
            pl.BlockSpec((1, DH), lambda i: (0, 0)),
            pl.BlockSpec((1, DH), lambda i: (0, 0)),
            pl.BlockSpec((DH, D), lambda i: (0, 0)),
            pl.BlockSpec((1, D), lambda i: (0, 0)),
            pl.BlockSpec((D, D), lambda i: (0, 0)),
            pl.BlockSpec((1, D), lambda i: (0, 0)),
            pl.BlockSpec((D, D), lambda i: (0, 0)),
            pl.BlockSpec((1, D), lambda i: (0, 0)),
            pl.BlockSpec((D, D), lambda i: (0, 0)),
            pl.BlockSpec((1, D), lambda i: (0, 0)),
            pl.BlockSpec((20, D), lambda i: (0, 0)),
            pl.BlockSpec((20, D), lambda i: (0, 0)),
            pl.BlockSpec((20, D), lambda i: (0, 0)),
        ],
        out_specs=[
            pl.BlockSpec((MB, D), lambda i: (i, 0)),
            pl.BlockSpec((SUBB, D), lambda i: (i, 0)),
        ],
        out_shape=(
            jax.ShapeDtypeStruct((M, D), F32),
            jax.ShapeDtypeStruct((N, D), F32),
        ),
    )(agg, dinv, hop_col, stats, hopW32, row2(bn_gamma),
      row2(bn_beta), enc_W1, row2(enc_b1), enc_W2, row2(enc_b2), sub_W,
      row2(sub_b), ctx_W, row2(ctx_b), gcT, gsT, gxT)

    ctxp = _ctx_kernel(ctxall, map3)

    out = pl.pallas_call(
        _final_body,
        grid=(10,),
        in_specs=[
            pl.BlockSpec((1000, D), lambda i: (i, 0)),
            pl.BlockSpec((2, 1000, D), lambda i: (0, i, 0)),
            pl.BlockSpec((1000, 16), lambda i: (i, 0)),
            pl.BlockSpec((1000, 16), lambda i: (i, 0)),
        ],
        out_specs=pl.BlockSpec((1000, D), lambda i: (i, 0)),
        out_shape=jax.ShapeDtypeStruct((N, D), F32),
    )(outp, ctxp, cnt0, cnt1)

    return out
